# baseline (speedup 1.0000x reference)
import functools

import jax
import jax.numpy as jnp
from jax import lax
from jax.experimental import pallas as pl
from jax.experimental.pallas import tpu as pltpu
from jax.experimental.pallas import tpu_sc as plsc

VOCAB = 8192
D = 64
NTOK = 8 * 576


TT = 512
KB = 2048
N_T = NTOK // TT
N_K = VOCAB // KB


def _argmax_body(x_ref, w_ref, iota2_ref, idx_ref, max_s, idx_s, arg_s):
    k = pl.program_id(1)
    logits = jax.lax.dot_general(
        x_ref[...], w_ref[...],
        dimension_numbers=(((1,), (1,)), ((), ())),
        preferred_element_type=jnp.float32,
    )
    local_max = jnp.max(logits, axis=1, keepdims=True)
    onehot = jnp.where(logits == local_max, 1.0, 0.0)
    sums = jax.lax.dot_general(
        onehot, iota2_ref[...],
        dimension_numbers=(((1,), (0,)), ((), ())),
        preferred_element_type=jnp.float32,
    )
    hi = (sums[:, 0:1] + 0.25).astype(jnp.int32)
    lo = (sums[:, 1:2] + 0.25).astype(jnp.int32)
    arg_s[...] = hi * 16 + lo

    @pl.when(jnp.max(sums[:, 2:3]) > 1.5)
    def _tie_exact():
        col = lax.broadcasted_iota(jnp.int32, (TT, KB), 1)
        arg_s[...] = jnp.min(
            jnp.where(logits == local_max, col, VOCAB), axis=1, keepdims=True
        )

    local_arg = arg_s[...] + k * KB

    @pl.when(k == 0)
    def _init():
        max_s[...] = local_max
        idx_s[...] = local_arg

    @pl.when(k > 0)
    def _combine():
        better = local_max > max_s[...]
        idx_s[...] = jnp.where(better, local_arg, idx_s[...])
        max_s[...] = jnp.maximum(max_s[...], local_max)

    @pl.when(k == N_K - 1)
    def _emit():
        idx_ref[...] = idx_s[...]


def _tc_argmax(x, W):
    ar = jnp.arange(KB, dtype=jnp.int32)
    iota2 = jnp.zeros((KB, 128), jnp.float32)
    iota2 = iota2.at[:, 0].set((ar // 16).astype(jnp.float32))
    iota2 = iota2.at[:, 1].set((ar % 16).astype(jnp.float32))
    iota2 = iota2.at[:, 2].set(1.0)
    return pl.pallas_call(
        _argmax_body,
        grid=(N_T, N_K),
        in_specs=[
            pl.BlockSpec((TT, D), lambda t, k: (t, 0)),
            pl.BlockSpec((KB, D), lambda t, k: (k, 0)),
            pl.BlockSpec((KB, 128), lambda t, k: (0, 0)),
        ],
        out_specs=pl.BlockSpec((TT, 1), lambda t, k: (t, 0)),
        out_shape=jax.ShapeDtypeStruct((NTOK, 1), jnp.int32),
        scratch_shapes=[
            pltpu.VMEM((TT, 1), jnp.float32),
            pltpu.VMEM((TT, 1), jnp.int32),
            pltpu.VMEM((TT, 1), jnp.int32),
        ],
        compiler_params=pltpu.CompilerParams(
            dimension_semantics=("arbitrary", "arbitrary"),
        ),
    )(x, W, iota2)


_NW = 32
_BPW = NTOK // _NW

@functools.lru_cache(maxsize=1)
def _sc_gather_fn():
    mesh = plsc.VectorSubcoreMesh(core_axis_name="c", subcore_axis_name="s")

    @functools.partial(
        pl.kernel,
        out_type=jax.ShapeDtypeStruct((NTOK, D), jnp.float32),
        mesh=mesh,
        scratch_types=[
            pltpu.VMEM((_BPW,), jnp.int32),
            pltpu.VMEM((_BPW, D), jnp.float32),
            pltpu.SemaphoreType.DMA,
        ],
        compiler_params=pltpu.CompilerParams(use_tc_tiling_on_sc=False),
    )
    def _sc_gather(table_hbm, idx_hbm, out_hbm, idx_v, rows_v, sem):
        wid = lax.axis_index("s") * 2 + lax.axis_index("c")
        base = wid * _BPW
        pltpu.sync_copy(idx_hbm.at[pl.ds(base, _BPW)], idx_v)
        pltpu.async_copy(table_hbm.at[idx_v], rows_v, sem).wait()
        pltpu.sync_copy(rows_v, out_hbm.at[pl.ds(base, _BPW)])

    return _sc_gather


def kernel(input, W):
    x = input.reshape(NTOK, D)
    idxs = _tc_argmax(x, W).reshape(NTOK)
    quantize = _sc_gather_fn()(W, idxs)
    return quantize.reshape(8, 576, D), idxs.reshape(8, 576)


# speedup vs baseline: 1.9266x; 1.9266x over previous
import functools

import jax
import jax.numpy as jnp
from jax import lax
from jax.experimental import pallas as pl
from jax.experimental.pallas import tpu as pltpu
from jax.experimental.pallas import tpu_sc as plsc

VOCAB = 8192
D = 64
NTOK = 8 * 576


TT = 1152
KB = 8192
N_T = NTOK // TT
N_K = VOCAB // KB


def _argmax_body(x_ref, w_ref, idx_ref, max_s, idx_s):
    k = pl.program_id(1)
    logits = jax.lax.dot_general(
        x_ref[...], w_ref[...],
        dimension_numbers=(((1,), (1,)), ((), ())),
        preferred_element_type=jnp.float32,
    )
    local_max = jnp.max(logits, axis=1, keepdims=True)
    col = lax.broadcasted_iota(jnp.int32, (TT, KB), 1)
    local_arg = jnp.min(
        jnp.where(logits == local_max, col, VOCAB), axis=1, keepdims=True
    ) + k * KB

    @pl.when(k == 0)
    def _init():
        max_s[...] = local_max
        idx_s[...] = local_arg

    @pl.when(k > 0)
    def _combine():
        better = local_max > max_s[...]
        idx_s[...] = jnp.where(better, local_arg, idx_s[...])
        max_s[...] = jnp.maximum(max_s[...], local_max)

    @pl.when(k == N_K - 1)
    def _emit():
        idx_ref[...] = idx_s[...]


def _tc_argmax(x, W):
    return pl.pallas_call(
        _argmax_body,
        grid=(N_T, N_K),
        in_specs=[
            pl.BlockSpec((TT, D), lambda t, k: (t, 0)),
            pl.BlockSpec((KB, D), lambda t, k: (k, 0)),
        ],
        out_specs=pl.BlockSpec((TT, 1), lambda t, k: (t, 0)),
        out_shape=jax.ShapeDtypeStruct((NTOK, 1), jnp.int32),
        scratch_shapes=[
            pltpu.VMEM((TT, 1), jnp.float32),
            pltpu.VMEM((TT, 1), jnp.int32),
        ],
        compiler_params=pltpu.CompilerParams(
            dimension_semantics=("arbitrary", "arbitrary"),
        ),
    )(x, W)


_NW = 32
_BPW = NTOK // _NW

@functools.lru_cache(maxsize=1)
def _sc_gather_fn():
    mesh = plsc.VectorSubcoreMesh(core_axis_name="c", subcore_axis_name="s")

    @functools.partial(
        pl.kernel,
        out_type=jax.ShapeDtypeStruct((NTOK, D), jnp.float32),
        mesh=mesh,
        scratch_types=[
            pltpu.VMEM((_BPW,), jnp.int32),
            pltpu.VMEM((_BPW, D), jnp.float32),
            pltpu.SemaphoreType.DMA,
        ],
        compiler_params=pltpu.CompilerParams(use_tc_tiling_on_sc=False),
    )
    def _sc_gather(table_hbm, idx_hbm, out_hbm, idx_v, rows_v, sem):
        wid = lax.axis_index("s") * 2 + lax.axis_index("c")
        base = wid * _BPW
        pltpu.sync_copy(idx_hbm.at[pl.ds(base, _BPW)], idx_v)
        pltpu.async_copy(table_hbm.at[idx_v], rows_v, sem).wait()
        pltpu.sync_copy(rows_v, out_hbm.at[pl.ds(base, _BPW)])

    return _sc_gather


def kernel(input, W):
    x = input.reshape(NTOK, D)
    idxs = _tc_argmax(x, W).reshape(NTOK)
    quantize = _sc_gather_fn()(W, idxs)
    return quantize.reshape(8, 576, D), idxs.reshape(8, 576)


# speedup vs baseline: 2.0362x; 1.0569x over previous
import functools

import jax
import jax.numpy as jnp
from jax import lax
from jax.experimental import pallas as pl
from jax.experimental.pallas import tpu as pltpu
from jax.experimental.pallas import tpu_sc as plsc

VOCAB = 8192
D = 64
NTOK = 8 * 576


TT = 1152
KB = 8192
N_T = NTOK // TT
N_K = VOCAB // KB


def _argmax_body(x_ref, w_ref, idx_ref, max_s, idx_s):
    k = pl.program_id(1)
    logits = jax.lax.dot_general(
        x_ref[...], w_ref[...],
        dimension_numbers=(((1,), (1,)), ((), ())),
        preferred_element_type=jnp.float32,
    )
    local_max = jnp.max(logits, axis=1, keepdims=True)
    col = lax.broadcasted_iota(jnp.int32, (TT, KB), 1).astype(jnp.float32)
    local_arg = jnp.min(
        jnp.where(logits == local_max, col, float(VOCAB)), axis=1, keepdims=True
    ).astype(jnp.int32) + k * KB

    @pl.when(k == 0)
    def _init():
        max_s[...] = local_max
        idx_s[...] = local_arg

    @pl.when(k > 0)
    def _combine():
        better = local_max > max_s[...]
        idx_s[...] = jnp.where(better, local_arg, idx_s[...])
        max_s[...] = jnp.maximum(max_s[...], local_max)

    @pl.when(k == N_K - 1)
    def _emit():
        idx_ref[...] = idx_s[...]


def _tc_argmax(x, W):
    return pl.pallas_call(
        _argmax_body,
        grid=(N_T, N_K),
        in_specs=[
            pl.BlockSpec((TT, D), lambda t, k: (t, 0)),
            pl.BlockSpec((KB, D), lambda t, k: (k, 0)),
        ],
        out_specs=pl.BlockSpec((TT, 1), lambda t, k: (t, 0)),
        out_shape=jax.ShapeDtypeStruct((NTOK, 1), jnp.int32),
        scratch_shapes=[
            pltpu.VMEM((TT, 1), jnp.float32),
            pltpu.VMEM((TT, 1), jnp.int32),
        ],
        compiler_params=pltpu.CompilerParams(
            dimension_semantics=("arbitrary", "arbitrary"),
        ),
    )(x, W)


_NW = 32
_BPW = NTOK // _NW

@functools.lru_cache(maxsize=1)
def _sc_gather_fn():
    mesh = plsc.VectorSubcoreMesh(core_axis_name="c", subcore_axis_name="s")

    @functools.partial(
        pl.kernel,
        out_type=jax.ShapeDtypeStruct((NTOK, D), jnp.float32),
        mesh=mesh,
        scratch_types=[
            pltpu.VMEM((_BPW,), jnp.int32),
            pltpu.VMEM((_BPW, D), jnp.float32),
            pltpu.SemaphoreType.DMA,
        ],
        compiler_params=pltpu.CompilerParams(use_tc_tiling_on_sc=False),
    )
    def _sc_gather(table_hbm, idx_hbm, out_hbm, idx_v, rows_v, sem):
        wid = lax.axis_index("s") * 2 + lax.axis_index("c")
        base = wid * _BPW
        pltpu.sync_copy(idx_hbm.at[pl.ds(base, _BPW)], idx_v)
        pltpu.async_copy(table_hbm.at[idx_v], rows_v, sem).wait()
        pltpu.sync_copy(rows_v, out_hbm.at[pl.ds(base, _BPW)])

    return _sc_gather


def kernel(input, W):
    x = input.reshape(NTOK, D)
    idxs = _tc_argmax(x, W).reshape(NTOK)
    quantize = _sc_gather_fn()(W, idxs)
    return quantize.reshape(8, 576, D), idxs.reshape(8, 576)
